# Initial kernel scaffold; baseline (speedup 1.0000x reference)
import jax
import jax.numpy as jnp
from jax.experimental import pallas as pl


def kernel(M, params, sw_params, kinds, time):
    raise NotImplementedError("write your pallas kernel here")


# speedup vs baseline: 5.1842x; 5.1842x over previous
import jax
import jax.numpy as jnp
from jax.experimental import pallas as pl

E = 2048
N = 1024
OUT = 2 * E + N
DT = 1e-06

R = 256
NB = (N + 2 * E) // R
B1 = N // R
B2 = (N + E) // R
MT_NB = E // R


def _band_kernel(m_ref, mt_ref, p_ref, k_ref, s_ref, out_ref):
    i = pl.program_id(0)

    @pl.when(i < B1)
    def _kcl():
        out_ref[:, 0:E] = m_ref[...]
        out_ref[:, E:] = jnp.zeros((R, OUT - E), jnp.float32)

    @pl.when(jnp.logical_and(i >= B1, i < B2))
    def _kvl():
        e0 = (i - B1) * R
        rows = jax.lax.broadcasted_iota(jnp.int32, (R, E), 0)
        cols = jax.lax.broadcasted_iota(jnp.int32, (R, E), 1)
        out_ref[:, 0:E] = jnp.zeros((R, E), jnp.float32)
        out_ref[:, E:2 * E] = jnp.where(cols == rows + e0, 1.0, 0.0)
        out_ref[:, 2 * E:] = -mt_ref[...].T

    @pl.when(i >= B2)
    def _el():
        e0 = (i - B2) * R
        params = p_ref[...]
        kinds = k_ref[...]
        sw_on = s_ref[...] > 0.0
        z = jnp.where(kinds == 0, -params,
            jnp.where(kinds == 4, -DT / params,
            jnp.where(kinds == 5, 1.0,
            jnp.where(kinds == 2, 1.0,
            jnp.where(jnp.logical_and(kinds == 3, jnp.logical_not(sw_on)),
                      1.0, 0.0)))))
        y = jnp.where(kinds == 0, 1.0,
            jnp.where(kinds == 4, 1.0,
            jnp.where(kinds == 5, -DT / params,
            jnp.where(kinds == 1, 1.0,
            jnp.where(jnp.logical_and(kinds == 3, sw_on), 1.0, 0.0)))))
        rows = jax.lax.broadcasted_iota(jnp.int32, (R, E), 0)
        cols = jax.lax.broadcasted_iota(jnp.int32, (R, E), 1)
        diag = cols == rows + e0
        out_ref[:, 0:E] = jnp.where(diag, z, 0.0)
        out_ref[:, E:2 * E] = jnp.where(diag, y, 0.0)
        out_ref[:, 2 * E:] = jnp.zeros((R, N), jnp.float32)


def kernel(M, params, sw_params, kinds, time):
    swcol = sw_params[:, time]
    p2 = params.reshape(1, E).astype(jnp.float32)
    k2 = kinds.reshape(1, E).astype(jnp.int32)
    s2 = swcol.reshape(1, E).astype(jnp.float32)

    grid = (NB,)
    out = pl.pallas_call(
        _band_kernel,
        grid=grid,
        in_specs=[
            pl.BlockSpec((R, E), lambda i: (jnp.minimum(i, B1 - 1), 0)),
            pl.BlockSpec((N, R), lambda i: (0, jnp.clip(i - B1, 0, MT_NB - 1))),
            pl.BlockSpec((1, E), lambda i: (0, 0)),
            pl.BlockSpec((1, E), lambda i: (0, 0)),
            pl.BlockSpec((1, E), lambda i: (0, 0)),
        ],
        out_specs=pl.BlockSpec((R, OUT), lambda i: (i, 0)),
        out_shape=jax.ShapeDtypeStruct((OUT, OUT), jnp.float32),
    )(M, M, p2, k2, s2)
    return out


# speedup vs baseline: 5.5080x; 1.0625x over previous
import jax
import jax.numpy as jnp
from jax.experimental import pallas as pl

E = 2048
N = 1024
OUT = 2 * E + N
DT = 1e-06

R = 512
NB = (N + 2 * E) // R
B1 = N // R
B2 = (N + E) // R
MT_NB = E // R


def _band_kernel(m_ref, mt_ref, p_ref, k_ref, s_ref, out_ref):
    i = pl.program_id(0)

    @pl.when(i < B1)
    def _kcl():
        out_ref[:, 0:E] = m_ref[...]
        out_ref[:, E:] = jnp.zeros((R, OUT - E), jnp.float32)

    @pl.when(jnp.logical_and(i >= B1, i < B2))
    def _kvl():
        e0 = (i - B1) * R
        rows = jax.lax.broadcasted_iota(jnp.int32, (R, E), 0)
        cols = jax.lax.broadcasted_iota(jnp.int32, (R, E), 1)
        out_ref[:, 0:E] = jnp.zeros((R, E), jnp.float32)
        out_ref[:, E:2 * E] = jnp.where(cols == rows + e0, 1.0, 0.0)
        out_ref[:, 2 * E:] = -mt_ref[...].T

    @pl.when(i >= B2)
    def _el():
        e0 = (i - B2) * R
        params = p_ref[...]
        kinds = k_ref[...]
        sw_on = s_ref[...] > 0.0
        z = jnp.where(kinds == 0, -params,
            jnp.where(kinds == 4, -DT / params,
            jnp.where(kinds == 5, 1.0,
            jnp.where(kinds == 2, 1.0,
            jnp.where(jnp.logical_and(kinds == 3, jnp.logical_not(sw_on)),
                      1.0, 0.0)))))
        y = jnp.where(kinds == 0, 1.0,
            jnp.where(kinds == 4, 1.0,
            jnp.where(kinds == 5, -DT / params,
            jnp.where(kinds == 1, 1.0,
            jnp.where(jnp.logical_and(kinds == 3, sw_on), 1.0, 0.0)))))
        rows = jax.lax.broadcasted_iota(jnp.int32, (R, E), 0)
        cols = jax.lax.broadcasted_iota(jnp.int32, (R, E), 1)
        diag = cols == rows + e0
        out_ref[:, 0:E] = jnp.where(diag, z, 0.0)
        out_ref[:, E:2 * E] = jnp.where(diag, y, 0.0)
        out_ref[:, 2 * E:] = jnp.zeros((R, N), jnp.float32)


def kernel(M, params, sw_params, kinds, time):
    swcol = sw_params[:, time]
    p2 = params.reshape(1, E).astype(jnp.float32)
    k2 = kinds.reshape(1, E).astype(jnp.int32)
    s2 = swcol.reshape(1, E).astype(jnp.float32)

    grid = (NB,)
    out = pl.pallas_call(
        _band_kernel,
        grid=grid,
        in_specs=[
            pl.BlockSpec((R, E), lambda i: (jnp.minimum(i, B1 - 1), 0)),
            pl.BlockSpec((N, R), lambda i: (0, jnp.clip(i - B1, 0, MT_NB - 1))),
            pl.BlockSpec((1, E), lambda i: (0, 0)),
            pl.BlockSpec((1, E), lambda i: (0, 0)),
            pl.BlockSpec((1, E), lambda i: (0, 0)),
        ],
        out_specs=pl.BlockSpec((R, OUT), lambda i: (i, 0)),
        out_shape=jax.ShapeDtypeStruct((OUT, OUT), jnp.float32),
    )(M, M, p2, k2, s2)
    return out
